# baseline (device time: 44602 ns/iter reference)
import jax
import jax.numpy as jnp
from jax import lax
from jax.experimental import pallas as pl
from jax.experimental.pallas import tpu as pltpu

N_DEV = 16
M_BLK = 256
PLANES = 4
PER_PLANE = N_DEV // PLANES

_ORDERS = {0: (0, 1, 2, 3), 1: (1, 0, 2, 3), 2: (2, 3, 1, 0), 3: (3, 2, 1, 0)}
_SEND_ORDER = (8, 9, 7, 10, 6, 11, 5, 12, 4, 13, 3, 14, 2, 15, 1)


def kernel(x, w_mat):
    k_total, m_shard = x.shape
    n = w_mat.shape[1]
    kw = k_total // PLANES

    def body(x_ref, w_ref, out_ref, xq_ref, xg_ref, xd_ref, sc_send_ref,
             sc_recv_ref, amax_ref, send_sems, recv_sems, sc_send_sems,
             sc_recv_sems, amax_send_sems, amax_recv_sems):
        my = lax.axis_index("i")
        my_plane = my // PER_PLANE

        barrier_sem = pltpu.get_barrier_semaphore()
        for d in range(1, N_DEV):
            pl.semaphore_signal(
                barrier_sem, inc=1,
                device_id=((my + d) % N_DEV,),
                device_id_type=pl.DeviceIdType.MESH,
            )

        for j in range(N_DEV):
            blk = x_ref[pl.ds(j * M_BLK, M_BLK), :]
            scale = jnp.max(jnp.abs(blk)) / 127.0
            q = jnp.clip(jnp.round(blk / scale), -127.0, 127.0)
            xq_ref[pl.ds(j * M_BLK, M_BLK), :] = q.astype(jnp.int8)
            sc_send_ref[j, :, :] = jnp.full((8, 128), scale, dtype=jnp.float32)

        def send_to(d, dst):
            blk_rdma = pltpu.make_async_remote_copy(
                src_ref=xq_ref.at[pl.ds(dst * M_BLK, M_BLK), :],
                dst_ref=xg_ref.at[:, pl.ds(my * M_BLK, M_BLK)],
                send_sem=send_sems.at[d],
                recv_sem=recv_sems.at[my],
                device_id=(dst,),
                device_id_type=pl.DeviceIdType.MESH,
            )
            blk_rdma.start()
            sc_rdma = pltpu.make_async_remote_copy(
                src_ref=sc_send_ref.at[dst],
                dst_ref=sc_recv_ref.at[my],
                send_sem=sc_send_sems.at[d],
                recv_sem=sc_recv_sems.at[my],
                device_id=(dst,),
                device_id_type=pl.DeviceIdType.MESH,
            )
            sc_rdma.start()
            return (blk_rdma, sc_rdma)

        sends = list(send_to(0, my))

        pl.semaphore_wait(barrier_sem, N_DEV - 1)

        for d in _SEND_ORDER:
            sends.extend(send_to(d, (my + d) % N_DEV))

        def wait_and_dequant(s):
            win = pl.ds(s * M_BLK, M_BLK)
            blk_wait = pltpu.make_async_remote_copy(
                src_ref=xg_ref.at[:, win],
                dst_ref=xg_ref.at[:, win],
                send_sem=send_sems.at[0],
                recv_sem=recv_sems.at[s],
                device_id=(0,),
                device_id_type=pl.DeviceIdType.MESH,
            )
            blk_wait.wait_recv()
            sc_wait = pltpu.make_async_remote_copy(
                src_ref=sc_recv_ref.at[s],
                dst_ref=sc_recv_ref.at[s],
                send_sem=send_sems.at[0],
                recv_sem=sc_recv_sems.at[s],
                device_id=(0,),
                device_id_type=pl.DeviceIdType.MESH,
            )
            sc_wait.wait_recv()
            sc = sc_recv_ref[s, 0, 0].astype(jnp.bfloat16)
            xd_ref[:, win] = xg_ref[:, win].astype(jnp.bfloat16) * sc

        def chunk_dot(z, accumulate):
            for s in range(z * PER_PLANE, (z + 1) * PER_PLANE):
                wait_and_dequant(s)
            partial = jnp.dot(
                xd_ref[:, pl.ds(z * kw, kw)],
                w_ref[pl.ds(z * kw, kw), :],
                preferred_element_type=jnp.float32,
            )
            if accumulate:
                out_ref[:, :] += partial
            else:
                out_ref[:, :] = partial

        for q in range(PLANES):
            @pl.when(my_plane == q)
            def _(q=q):
                for idx, z in enumerate(_ORDERS[q]):
                    chunk_dot(z, accumulate=(idx > 0))

        for rdma in sends:
            rdma.wait_send()

        y = jnp.maximum(out_ref[:, :], 0.0)
        out_ref[:, :] = y
        amax_ref[0, :, :] = jnp.full((8, 128), jnp.max(y), dtype=jnp.float32)

        amax_sends = []
        for d in range(1, N_DEV):
            dst = (my + d) % N_DEV
            rdma = pltpu.make_async_remote_copy(
                src_ref=amax_ref.at[0],
                dst_ref=amax_ref.at[d],
                send_sem=amax_send_sems.at[d],
                recv_sem=amax_recv_sems.at[d],
                device_id=(dst,),
                device_id_type=pl.DeviceIdType.MESH,
            )
            rdma.start()
            amax_sends.append(rdma)
        for rdma in amax_sends:
            rdma.wait_recv()
        for rdma in amax_sends:
            rdma.wait_send()

        scale = jnp.max(amax_ref[:, :, :]) / 127.0
        q8 = jnp.clip(jnp.round(out_ref[:, :] / scale), -127.0, 127.0)
        out_ref[:, :] = q8 * scale

    return pl.pallas_call(
        body,
        out_shape=jax.ShapeDtypeStruct((M_BLK, n), jnp.float32),
        in_specs=[
            pl.BlockSpec(memory_space=pltpu.VMEM),
            pl.BlockSpec(memory_space=pltpu.VMEM),
        ],
        out_specs=pl.BlockSpec(memory_space=pltpu.VMEM),
        scratch_shapes=[
            pltpu.VMEM((k_total, m_shard), jnp.int8),
            pltpu.VMEM((M_BLK, k_total), jnp.int8),
            pltpu.VMEM((M_BLK, k_total), jnp.bfloat16),
            pltpu.VMEM((N_DEV, 8, 128), jnp.float32),
            pltpu.VMEM((N_DEV, 8, 128), jnp.float32),
            pltpu.VMEM((N_DEV, 8, 128), jnp.float32),
            pltpu.SemaphoreType.DMA((N_DEV,)),
            pltpu.SemaphoreType.DMA((N_DEV,)),
            pltpu.SemaphoreType.DMA((N_DEV,)),
            pltpu.SemaphoreType.DMA((N_DEV,)),
            pltpu.SemaphoreType.DMA((N_DEV,)),
            pltpu.SemaphoreType.DMA((N_DEV,)),
        ],
        compiler_params=pltpu.CompilerParams(
            collective_id=0,
            vmem_limit_bytes=100 * 1024 * 1024,
        ),
    )(x, w_mat)
